# baseline (speedup 1.0000x reference)
import dataclasses

import jax
import jax.numpy as jnp
from jax import lax
from jax.experimental import pallas as pl
from jax.experimental.pallas import tpu as pltpu
from jax.experimental.pallas import tpu_sc as plsc

NC = 2
NS = 16
NW = NC * NS
LANES = 16
CHUNK = 96
PAD_ROWS = 8


def _make_sc_aggregate(n, e_pad, d, with_deg):
  ew = e_pad // NW
  nch = ew // CHUNK
  assert ew % CHUNK == 0
  assert nch % 2 == 1 and nch >= 3
  nt = (n // NS) // 8 * 8
  rem = n - NS * nt
  mesh = plsc.VectorSubcoreMesh(core_axis_name="c", subcore_axis_name="s")

  out_type = [jax.ShapeDtypeStruct((NC, n, d), jnp.float32)]
  if with_deg:
    out_type.append(jax.ShapeDtypeStruct((NW, 1, n), jnp.float32))

  scratch = [
      pltpu.VMEM((CHUNK,), jnp.int32),
      pltpu.VMEM((CHUNK,), jnp.int32),
      pltpu.VMEM((CHUNK,), jnp.int32),
      pltpu.VMEM((CHUNK,), jnp.int32),
      pltpu.VMEM((CHUNK, d), jnp.float32),
      pltpu.VMEM((CHUNK, d), jnp.float32),
      pltpu.VMEM_SHARED((n + PAD_ROWS, d), jnp.float32),
      pltpu.SemaphoreType.DMA,
      pltpu.SemaphoreType.DMA,
      pltpu.SemaphoreType.DMA,
      pltpu.SemaphoreType.DMA,
  ]
  if with_deg:
    scratch.append(
        pltpu.VMEM((n + PAD_ROWS,), jnp.float32))

  def body(feats_hbm, src_hbm, dst_hbm, *refs):
    if with_deg:
      (out_hbm, deg_hbm, srcv0, srcv1, dstv0, dstv1, rows0, rows1, acc,
       gsem0, gsem1, isem0, isem1, degloc) = refs
    else:
      (out_hbm, srcv0, srcv1, dstv0, dstv1, rows0, rows1, acc,
       gsem0, gsem1, isem0, isem1) = refs
    srcv = (srcv0, srcv1)
    dstv = (dstv0, dstv1)
    rows = (rows0, rows1)
    gsem = (gsem0, gsem1)
    isem = (isem0, isem1)

    cid = lax.axis_index("c")
    tid = lax.axis_index("s")
    wid = cid * NS + tid

    zero16 = jnp.zeros((LANES,), jnp.float32)

    @pl.loop(0, CHUNK)
    def _(r):
      for k in range(d // LANES):
        rows0[r, pl.ds(k * LANES, LANES)] = zero16

    row0 = tid * nt
    for j in range(nt // CHUNK):
      pltpu.sync_copy(rows0, acc.at[pl.ds(row0 + j * CHUNK, CHUNK)])
    tail = nt % CHUNK
    if tail:
      pltpu.sync_copy(rows0.at[pl.ds(0, tail)],
                      acc.at[pl.ds(row0 + nt - tail, tail)])

    @pl.when(tid == NS - 1)
    def _():
      pltpu.sync_copy(rows0.at[pl.ds(0, rem)], acc.at[pl.ds(NS * nt, rem)])

    if with_deg:
      @pl.loop(0, n // LANES)
      def _(i):
        degloc[pl.ds(i * LANES, LANES)] = zero16

    plsc.subcore_barrier()

    ones16 = jnp.full((LANES,), 1.0, jnp.float32)
    base_w = wid * ew

    def idx_descs(c, b):
      base = base_w + c * CHUNK
      return (
          pltpu.make_async_copy(src_hbm.at[pl.ds(base, CHUNK)], srcv[b],
                                isem[b]),
          pltpu.make_async_copy(dst_hbm.at[pl.ds(base, CHUNK)], dstv[b],
                                isem[b]),
      )

    def idx_start(c, b):
      for d_ in idx_descs(c, b):
        d_.start()

    def idx_wait(c, b):
      for d_ in idx_descs(c, b):
        d_.wait()

    def start_gather(b):
      pltpu.make_async_copy(feats_hbm.at[srcv[b]], rows[b], gsem[b]).start()

    def wait_gather(b):
      pltpu.make_async_copy(feats_hbm.at[srcv[b]], rows[b], gsem[b]).wait()

    def scatter(b):
      pltpu.sync_copy(rows[b], acc.at[dstv[b]], add=True)

    def deg_update(b):
      if with_deg:
        for j in range(CHUNK // LANES):
          idx = dstv[b][pl.ds(j * LANES, LANES)]
          plsc.addupdate_scatter(degloc, [idx], ones16)

    idx_start(0, 0)
    idx_start(1, 1)
    idx_wait(0, 0)
    start_gather(0)

    @pl.loop(0, (nch - 1) // 2)
    def _(i):
      c = 2 * i
      idx_wait(c + 1, 1)
      wait_gather(0)
      start_gather(1)
      scatter(0)
      deg_update(0)
      idx_start(c + 2, 0)

      idx_wait(c + 2, 0)
      wait_gather(1)
      start_gather(0)
      scatter(1)
      deg_update(1)

      @pl.when(c + 3 < nch)
      def _():
        idx_start(c + 3, 1)

    wait_gather(0)
    scatter(0)
    deg_update(0)

    plsc.subcore_barrier()

    pltpu.sync_copy(acc.at[pl.ds(row0, nt)], out_hbm.at[cid, pl.ds(row0, nt)])

    @pl.when(tid == NS - 1)
    def _():
      pltpu.sync_copy(acc.at[pl.ds(NS * nt, rem)],
                      out_hbm.at[cid, pl.ds(NS * nt, rem)])

    if with_deg:
      pltpu.sync_copy(degloc.at[pl.ds(0, n)], deg_hbm.at[wid, 0])

  cp = pltpu.CompilerParams()
  if "needs_layout_passes" in pltpu.CompilerParams.__dataclass_fields__:
    cp = dataclasses.replace(cp, needs_layout_passes=False)
  if d % 128 != 0:
    cp = dataclasses.replace(cp, use_tc_tiling_on_sc=False)
  return pl.kernel(body, out_type=out_type, mesh=mesh, scratch_types=scratch,
                   compiler_params=cp)


def _tc_layer1(agg_part, deg_part, w1, b1, w2):
  n = agg_part.shape[1]

  def body(aggp_ref, degp_ref, w1_ref, b1_ref, w2_ref, h2_ref, deg_ref):
    dp = degp_ref[...]
    deg = jnp.maximum(jnp.sum(dp, axis=(0, 1)), 1.0)
    deg_col = deg[:, None]
    deg_ref[...] = deg_col
    agg = aggp_ref[0] + aggp_ref[1]
    h = agg / deg_col
    h = jnp.dot(h, w1_ref[...], preferred_element_type=jnp.float32)
    h = jnp.maximum(h + b1_ref[...], 0.0)
    h2_ref[...] = jnp.dot(h, w2_ref[...], preferred_element_type=jnp.float32)

  return pl.pallas_call(
      body,
      out_shape=[
          jax.ShapeDtypeStruct((n, w2.shape[1]), jnp.float32),
          jax.ShapeDtypeStruct((n, 1), jnp.float32),
      ],
  )(agg_part, deg_part, w1, b1, w2)


def _tc_layer2(agg_part, deg_col, b2):
  n = agg_part.shape[1]

  def body(aggp_ref, deg_ref, b2_ref, out_ref):
    agg = aggp_ref[0] + aggp_ref[1]
    out_ref[...] = agg / deg_ref[...] + b2_ref[...]

  return pl.pallas_call(
      body,
      out_shape=jax.ShapeDtypeStruct((n, agg_part.shape[2]), jnp.float32),
  )(agg_part, deg_col, b2)


@jax.jit
def kernel(x, edge_index, W1, b1, W2, b2):
  n, d_in = x.shape
  e = edge_index.shape[1]
  ew = e // NW
  ew_pad = -(-ew // CHUNK) * CHUNK
  pad = ew_pad - ew

  def pad_edges(arr, fill):
    arr = arr.astype(jnp.int32).reshape(NW, ew)
    filler = jnp.full((NW, pad), fill, jnp.int32)
    return jnp.concatenate([arr, filler], axis=1).reshape(-1)

  src = pad_edges(edge_index[0], 0)
  dst = pad_edges(edge_index[1], n)
  e_pad = NW * ew_pad

  agg1_part, deg_part = _make_sc_aggregate(n, e_pad, d_in, True)(x, src, dst)
  h2, deg_col = _tc_layer1(agg1_part, deg_part, W1, b1.reshape(1, -1), W2)
  (agg2_part,) = _make_sc_aggregate(n, e_pad, h2.shape[1], False)(h2, src, dst)
  return _tc_layer2(agg2_part, deg_col, b2.reshape(1, -1))


# speedup vs baseline: 1.4206x; 1.4206x over previous
import dataclasses

import jax
import jax.numpy as jnp
from jax import lax
from jax.experimental import pallas as pl
from jax.experimental.pallas import tpu as pltpu
from jax.experimental.pallas import tpu_sc as plsc

NC = 2
NS = 16
NW = NC * NS
LANES = 16
CHUNK = 96
PAD_ROWS = 16


def _make_sc_aggregate(n, e_pad, d, with_deg):
  ew = e_pad // NW
  nch = ew // CHUNK
  assert ew % CHUNK == 0
  assert nch % 2 == 1 and nch >= 3
  nt = (n // NS) // 8 * 8
  rem = n - NS * nt
  mesh = plsc.VectorSubcoreMesh(core_axis_name="c", subcore_axis_name="s")

  out_type = [jax.ShapeDtypeStruct((NC, n, d), jnp.float32)]
  if with_deg:
    out_type.append(jax.ShapeDtypeStruct((NW, 1, n), jnp.float32))

  scratch = [
      pltpu.VMEM((CHUNK,), jnp.int32),
      pltpu.VMEM((CHUNK,), jnp.int32),
      pltpu.VMEM((CHUNK,), jnp.int32),
      pltpu.VMEM((CHUNK,), jnp.int32),
      pltpu.VMEM((CHUNK, d), jnp.float32),
      pltpu.VMEM((CHUNK, d), jnp.float32),
      pltpu.VMEM_SHARED((n + PAD_ROWS, d), jnp.float32),
      pltpu.SemaphoreType.DMA,
      pltpu.SemaphoreType.DMA,
      pltpu.SemaphoreType.DMA,
      pltpu.SemaphoreType.DMA,
  ]
  if with_deg:
    scratch.append(
        pltpu.VMEM((n + PAD_ROWS,), jnp.float32))

  def body(feats_hbm, src_hbm, dst_hbm, *refs):
    if with_deg:
      (out_hbm, deg_hbm, srcv0, srcv1, dstv0, dstv1, rows0, rows1, acc,
       gsem0, gsem1, isem0, isem1, degloc) = refs
    else:
      (out_hbm, srcv0, srcv1, dstv0, dstv1, rows0, rows1, acc,
       gsem0, gsem1, isem0, isem1) = refs
    srcv = (srcv0, srcv1)
    dstv = (dstv0, dstv1)
    rows = (rows0, rows1)
    gsem = (gsem0, gsem1)
    isem = (isem0, isem1)

    cid = lax.axis_index("c")
    tid = lax.axis_index("s")
    wid = cid * NS + tid

    zero16 = jnp.zeros((LANES,), jnp.float32)

    @pl.loop(0, CHUNK)
    def _(r):
      for k in range(d // LANES):
        rows0[r, pl.ds(k * LANES, LANES)] = zero16

    row0 = tid * nt
    for j in range(nt // CHUNK):
      pltpu.sync_copy(rows0, acc.at[pl.ds(row0 + j * CHUNK, CHUNK)])
    tail = nt % CHUNK
    if tail:
      pltpu.sync_copy(rows0.at[pl.ds(0, tail)],
                      acc.at[pl.ds(row0 + nt - tail, tail)])

    @pl.when(tid == NS - 1)
    def _():
      pltpu.sync_copy(rows0.at[pl.ds(0, rem)], acc.at[pl.ds(NS * nt, rem)])

    if with_deg:
      @pl.loop(0, n // LANES)
      def _(i):
        degloc[pl.ds(i * LANES, LANES)] = zero16

    plsc.subcore_barrier()

    ones16 = jnp.full((LANES,), 1.0, jnp.float32)
    base_w = wid * ew

    def idx_descs(c, b):
      base = base_w + c * CHUNK
      return (
          pltpu.make_async_copy(src_hbm.at[pl.ds(base, CHUNK)], srcv[b],
                                isem[b]),
          pltpu.make_async_copy(dst_hbm.at[pl.ds(base, CHUNK)], dstv[b],
                                isem[b]),
      )

    def idx_start(c, b):
      for d_ in idx_descs(c, b):
        d_.start()

    def idx_wait(c, b):
      for d_ in idx_descs(c, b):
        d_.wait()

    def start_gather(b):
      pltpu.make_async_copy(feats_hbm.at[srcv[b]], rows[b], gsem[b]).start()

    def wait_gather(b):
      pltpu.make_async_copy(feats_hbm.at[srcv[b]], rows[b], gsem[b]).wait()

    def scatter(b):
      pltpu.sync_copy(rows[b], acc.at[dstv[b]], add=True)

    def deg_update(b):
      if with_deg:
        for j in range(CHUNK // LANES):
          idx = dstv[b][pl.ds(j * LANES, LANES)]
          plsc.addupdate_scatter(degloc, [idx], ones16)

    idx_start(0, 0)
    idx_start(1, 1)
    idx_wait(0, 0)
    start_gather(0)

    @pl.loop(0, (nch - 1) // 2)
    def _(i):
      c = 2 * i
      idx_wait(c + 1, 1)
      wait_gather(0)
      start_gather(1)
      scatter(0)
      deg_update(0)
      idx_start(c + 2, 0)

      idx_wait(c + 2, 0)
      wait_gather(1)
      start_gather(0)
      scatter(1)
      deg_update(1)

      @pl.when(c + 3 < nch)
      def _():
        idx_start(c + 3, 1)

    wait_gather(0)
    scatter(0)
    deg_update(0)

    plsc.subcore_barrier()

    pltpu.sync_copy(acc.at[pl.ds(row0, nt)], out_hbm.at[cid, pl.ds(row0, nt)])

    @pl.when(tid == NS - 1)
    def _():
      pltpu.sync_copy(acc.at[pl.ds(NS * nt, rem)],
                      out_hbm.at[cid, pl.ds(NS * nt, rem)])

    if with_deg:
      pltpu.sync_copy(degloc.at[pl.ds(0, n)], deg_hbm.at[wid, 0])

  cp = pltpu.CompilerParams()
  if "needs_layout_passes" in pltpu.CompilerParams.__dataclass_fields__:
    cp = dataclasses.replace(cp, needs_layout_passes=False)
  if d % 128 != 0:
    cp = dataclasses.replace(cp, use_tc_tiling_on_sc=False)
  return pl.kernel(body, out_type=out_type, mesh=mesh, scratch_types=scratch,
                   compiler_params=cp)


def _tc_layer1(agg_part, deg_part, w1, b1, w2):
  n = agg_part.shape[1]

  def body(aggp_ref, degp_ref, w1_ref, b1_ref, w2_ref, h2_ref, deg_ref):
    dp = degp_ref[...]
    deg = jnp.maximum(jnp.sum(dp, axis=(0, 1)), 1.0)
    deg_col = deg[:, None]
    deg_ref[...] = deg_col
    agg = aggp_ref[0] + aggp_ref[1]
    h = agg / deg_col
    h = jnp.dot(h, w1_ref[...], preferred_element_type=jnp.float32)
    h = jnp.maximum(h + b1_ref[...], 0.0)
    h2_ref[...] = jnp.dot(h, w2_ref[...], preferred_element_type=jnp.float32)

  return pl.pallas_call(
      body,
      out_shape=[
          jax.ShapeDtypeStruct((n, w2.shape[1]), jnp.float32),
          jax.ShapeDtypeStruct((n, 1), jnp.float32),
      ],
  )(agg_part, deg_part, w1, b1, w2)


def _tc_layer2(agg_part, deg_col, b2):
  n = agg_part.shape[1]

  def body(aggp_ref, deg_ref, b2_ref, out_ref):
    agg = aggp_ref[0] + aggp_ref[1]
    out_ref[...] = agg / deg_ref[...] + b2_ref[...]

  return pl.pallas_call(
      body,
      out_shape=jax.ShapeDtypeStruct((n, agg_part.shape[2]), jnp.float32),
  )(agg_part, deg_col, b2)


@jax.jit
def kernel(x, edge_index, W1, b1, W2, b2):
  n, d_in = x.shape
  e = edge_index.shape[1]
  ew = e // NW
  ew_pad = -(-ew // CHUNK) * CHUNK
  pad = ew_pad - ew

  def pad_edges(arr, fill_row):
    arr = arr.astype(jnp.int32).reshape(NW, ew)
    filler = jnp.broadcast_to(fill_row[None, :], (NW, pad))
    return jnp.concatenate([arr, filler], axis=1).reshape(-1)

  idx = jnp.arange(pad, dtype=jnp.int32)
  src = pad_edges(edge_index[0], idx % 64)
  dst = pad_edges(edge_index[1], n + idx % PAD_ROWS)
  e_pad = NW * ew_pad

  agg1_part, deg_part = _make_sc_aggregate(n, e_pad, d_in, True)(x, src, dst)
  h2, deg_col = _tc_layer1(agg1_part, deg_part, W1, b1.reshape(1, -1), W2)
  (agg2_part,) = _make_sc_aggregate(n, e_pad, h2.shape[1], False)(h2, src, dst)
  return _tc_layer2(agg2_part, deg_col, b2.reshape(1, -1))


# speedup vs baseline: 1.5624x; 1.0998x over previous
import dataclasses

import jax
import jax.numpy as jnp
from jax import lax
from jax.experimental import pallas as pl
from jax.experimental.pallas import tpu as pltpu
from jax.experimental.pallas import tpu_sc as plsc

NC = 2
NS = 16
NW = NC * NS
LANES = 16
CHUNK = 128
PAD_ROWS = 16


def _make_sc_aggregate(n, e_pad, d, with_deg):
  ew = e_pad // NW
  nch = ew // CHUNK
  assert ew % CHUNK == 0
  assert nch % 2 == 1 and nch >= 3
  nt = (n // NS) // 8 * 8
  rem = n - NS * nt
  mesh = plsc.VectorSubcoreMesh(core_axis_name="c", subcore_axis_name="s")

  out_type = [jax.ShapeDtypeStruct((NC, n, d), jnp.float32)]
  if with_deg:
    out_type.append(jax.ShapeDtypeStruct((NW, 1, n), jnp.float32))

  scratch = [
      pltpu.VMEM((CHUNK,), jnp.int32),
      pltpu.VMEM((CHUNK,), jnp.int32),
      pltpu.VMEM((CHUNK,), jnp.int32),
      pltpu.VMEM((CHUNK,), jnp.int32),
      pltpu.VMEM((CHUNK, d), jnp.float32),
      pltpu.VMEM((CHUNK, d), jnp.float32),
      pltpu.VMEM_SHARED((n + PAD_ROWS, d), jnp.float32),
      pltpu.SemaphoreType.DMA,
      pltpu.SemaphoreType.DMA,
      pltpu.SemaphoreType.DMA,
      pltpu.SemaphoreType.DMA,
  ]
  if with_deg:
    scratch.append(
        pltpu.VMEM((n + PAD_ROWS,), jnp.float32))

  def body(feats_hbm, src_hbm, dst_hbm, *refs):
    if with_deg:
      (out_hbm, deg_hbm, srcv0, srcv1, dstv0, dstv1, rows0, rows1, acc,
       gsem0, gsem1, isem0, isem1, degloc) = refs
    else:
      (out_hbm, srcv0, srcv1, dstv0, dstv1, rows0, rows1, acc,
       gsem0, gsem1, isem0, isem1) = refs
    srcv = (srcv0, srcv1)
    dstv = (dstv0, dstv1)
    rows = (rows0, rows1)
    gsem = (gsem0, gsem1)
    isem = (isem0, isem1)

    cid = lax.axis_index("c")
    tid = lax.axis_index("s")
    wid = cid * NS + tid

    zero16 = jnp.zeros((LANES,), jnp.float32)

    @pl.loop(0, CHUNK)
    def _(r):
      for k in range(d // LANES):
        rows0[r, pl.ds(k * LANES, LANES)] = zero16

    row0 = tid * nt
    for j in range(nt // CHUNK):
      pltpu.sync_copy(rows0, acc.at[pl.ds(row0 + j * CHUNK, CHUNK)])
    tail = nt % CHUNK
    if tail:
      pltpu.sync_copy(rows0.at[pl.ds(0, tail)],
                      acc.at[pl.ds(row0 + nt - tail, tail)])

    @pl.when(tid == NS - 1)
    def _():
      pltpu.sync_copy(rows0.at[pl.ds(0, rem)], acc.at[pl.ds(NS * nt, rem)])

    if with_deg:
      @pl.loop(0, n // LANES)
      def _(i):
        degloc[pl.ds(i * LANES, LANES)] = zero16

    plsc.subcore_barrier()

    ones16 = jnp.full((LANES,), 1.0, jnp.float32)
    base_w = wid * ew

    def idx_descs(c, b):
      base = base_w + c * CHUNK
      return (
          pltpu.make_async_copy(src_hbm.at[pl.ds(base, CHUNK)], srcv[b],
                                isem[b]),
          pltpu.make_async_copy(dst_hbm.at[pl.ds(base, CHUNK)], dstv[b],
                                isem[b]),
      )

    def idx_start(c, b):
      for d_ in idx_descs(c, b):
        d_.start()

    def idx_wait(c, b):
      for d_ in idx_descs(c, b):
        d_.wait()

    def start_gather(b):
      pltpu.make_async_copy(feats_hbm.at[srcv[b]], rows[b], gsem[b]).start()

    def wait_gather(b):
      pltpu.make_async_copy(feats_hbm.at[srcv[b]], rows[b], gsem[b]).wait()

    def scatter(b):
      pltpu.sync_copy(rows[b], acc.at[dstv[b]], add=True)

    def deg_update(b):
      if with_deg:
        for j in range(CHUNK // LANES):
          idx = dstv[b][pl.ds(j * LANES, LANES)]
          plsc.addupdate_scatter(degloc, [idx], ones16)

    idx_start(0, 0)
    idx_start(1, 1)
    idx_wait(0, 0)
    start_gather(0)

    @pl.loop(0, (nch - 1) // 2)
    def _(i):
      c = 2 * i
      idx_wait(c + 1, 1)
      wait_gather(0)
      start_gather(1)
      scatter(0)
      deg_update(0)
      idx_start(c + 2, 0)

      idx_wait(c + 2, 0)
      wait_gather(1)
      start_gather(0)
      scatter(1)
      deg_update(1)

      @pl.when(c + 3 < nch)
      def _():
        idx_start(c + 3, 1)

    wait_gather(0)
    scatter(0)
    deg_update(0)

    plsc.subcore_barrier()

    pltpu.sync_copy(acc.at[pl.ds(row0, nt)], out_hbm.at[cid, pl.ds(row0, nt)])

    @pl.when(tid == NS - 1)
    def _():
      pltpu.sync_copy(acc.at[pl.ds(NS * nt, rem)],
                      out_hbm.at[cid, pl.ds(NS * nt, rem)])

    if with_deg:
      pltpu.sync_copy(degloc.at[pl.ds(0, n)], deg_hbm.at[wid, 0])

  cp = pltpu.CompilerParams()
  if "needs_layout_passes" in pltpu.CompilerParams.__dataclass_fields__:
    cp = dataclasses.replace(cp, needs_layout_passes=False)
  if d % 128 != 0:
    cp = dataclasses.replace(cp, use_tc_tiling_on_sc=False)
  return pl.kernel(body, out_type=out_type, mesh=mesh, scratch_types=scratch,
                   compiler_params=cp)


def _tc_layer1(agg_part, deg_part, w1, b1, w2):
  n = agg_part.shape[1]

  def body(aggp_ref, degp_ref, w1_ref, b1_ref, w2_ref, h2_ref, deg_ref):
    dp = degp_ref[...]
    deg = jnp.maximum(jnp.sum(dp, axis=(0, 1)), 1.0)
    deg_col = deg[:, None]
    deg_ref[...] = deg_col
    agg = aggp_ref[0] + aggp_ref[1]
    h = agg / deg_col
    h = jnp.dot(h, w1_ref[...], preferred_element_type=jnp.float32)
    h = jnp.maximum(h + b1_ref[...], 0.0)
    h2_ref[...] = jnp.dot(h, w2_ref[...], preferred_element_type=jnp.float32)

  return pl.pallas_call(
      body,
      out_shape=[
          jax.ShapeDtypeStruct((n, w2.shape[1]), jnp.float32),
          jax.ShapeDtypeStruct((n, 1), jnp.float32),
      ],
  )(agg_part, deg_part, w1, b1, w2)


def _tc_layer2(agg_part, deg_col, b2):
  n = agg_part.shape[1]

  def body(aggp_ref, deg_ref, b2_ref, out_ref):
    agg = aggp_ref[0] + aggp_ref[1]
    out_ref[...] = agg / deg_ref[...] + b2_ref[...]

  return pl.pallas_call(
      body,
      out_shape=jax.ShapeDtypeStruct((n, agg_part.shape[2]), jnp.float32),
  )(agg_part, deg_col, b2)


@jax.jit
def kernel(x, edge_index, W1, b1, W2, b2):
  n, d_in = x.shape
  e = edge_index.shape[1]
  ew = e // NW
  ew_pad = -(-ew // CHUNK) * CHUNK
  pad = ew_pad - ew

  def pad_edges(arr, fill_row):
    arr = arr.astype(jnp.int32).reshape(NW, ew)
    filler = jnp.broadcast_to(fill_row[None, :], (NW, pad))
    return jnp.concatenate([arr, filler], axis=1).reshape(-1)

  idx = jnp.arange(pad, dtype=jnp.int32)
  src = pad_edges(edge_index[0], idx % 64)
  dst = pad_edges(edge_index[1], n + idx % PAD_ROWS)
  e_pad = NW * ew_pad

  agg1_part, deg_part = _make_sc_aggregate(n, e_pad, d_in, True)(x, src, dst)
  h2, deg_col = _tc_layer1(agg1_part, deg_part, W1, b1.reshape(1, -1), W2)
  (agg2_part,) = _make_sc_aggregate(n, e_pad, h2.shape[1], False)(h2, src, dst)
  return _tc_layer2(agg2_part, deg_col, b2.reshape(1, -1))


# speedup vs baseline: 1.6821x; 1.0766x over previous
import dataclasses

import jax
import jax.numpy as jnp
from jax import lax
from jax.experimental import pallas as pl
from jax.experimental.pallas import tpu as pltpu
from jax.experimental.pallas import tpu_sc as plsc

NC = 2
NS = 16
NW = NC * NS
LANES = 16
CHUNK = 128
PAD_ROWS = 16
MSG_DTYPE = jnp.bfloat16


def _make_sc_aggregate(n, e_pad, d, with_deg, dtype=jnp.float32):
  ew = e_pad // NW
  nch = ew // CHUNK
  assert ew % CHUNK == 0
  assert nch % 2 == 1 and nch >= 3
  nt = (n // NS) // 8 * 8
  rem = n - NS * nt
  mesh = plsc.VectorSubcoreMesh(core_axis_name="c", subcore_axis_name="s")

  out_type = [jax.ShapeDtypeStruct((NC, n, d), dtype)]
  if with_deg:
    out_type.append(jax.ShapeDtypeStruct((NW, 1, n), jnp.float32))

  scratch = [
      pltpu.VMEM((CHUNK,), jnp.int32),
      pltpu.VMEM((CHUNK,), jnp.int32),
      pltpu.VMEM((CHUNK,), jnp.int32),
      pltpu.VMEM((CHUNK,), jnp.int32),
      pltpu.VMEM((CHUNK, d), dtype),
      pltpu.VMEM((CHUNK, d), dtype),
      pltpu.VMEM_SHARED((n + PAD_ROWS, d), dtype),
      pltpu.SemaphoreType.DMA,
      pltpu.SemaphoreType.DMA,
      pltpu.SemaphoreType.DMA,
      pltpu.SemaphoreType.DMA,
  ]
  if with_deg:
    scratch.append(
        pltpu.VMEM((n + PAD_ROWS,), jnp.float32))

  def body(feats_hbm, src_hbm, dst_hbm, *refs):
    if with_deg:
      (out_hbm, deg_hbm, srcv0, srcv1, dstv0, dstv1, rows0, rows1, acc,
       gsem0, gsem1, isem0, isem1, degloc) = refs
    else:
      (out_hbm, srcv0, srcv1, dstv0, dstv1, rows0, rows1, acc,
       gsem0, gsem1, isem0, isem1) = refs
    srcv = (srcv0, srcv1)
    dstv = (dstv0, dstv1)
    rows = (rows0, rows1)
    gsem = (gsem0, gsem1)
    isem = (isem0, isem1)

    cid = lax.axis_index("c")
    tid = lax.axis_index("s")
    wid = cid * NS + tid

    zero16 = jnp.zeros((LANES,), jnp.float32)
    vl = LANES * (4 // jnp.dtype(dtype).itemsize)
    zmsg = jnp.zeros((vl,), dtype)

    @pl.loop(0, CHUNK)
    def _(r):
      for k in range(d // vl):
        rows0[r, pl.ds(k * vl, vl)] = zmsg

    row0 = tid * nt
    for j in range(nt // CHUNK):
      pltpu.sync_copy(rows0, acc.at[pl.ds(row0 + j * CHUNK, CHUNK)])
    tail = nt % CHUNK
    if tail:
      pltpu.sync_copy(rows0.at[pl.ds(0, tail)],
                      acc.at[pl.ds(row0 + nt - tail, tail)])

    @pl.when(tid == NS - 1)
    def _():
      pltpu.sync_copy(rows0.at[pl.ds(0, rem)], acc.at[pl.ds(NS * nt, rem)])

    if with_deg:
      @pl.loop(0, n // LANES)
      def _(i):
        degloc[pl.ds(i * LANES, LANES)] = zero16

    plsc.subcore_barrier()

    ones16 = jnp.full((LANES,), 1.0, jnp.float32)
    base_w = wid * ew

    def idx_descs(c, b):
      base = base_w + c * CHUNK
      return (
          pltpu.make_async_copy(src_hbm.at[pl.ds(base, CHUNK)], srcv[b],
                                isem[b]),
          pltpu.make_async_copy(dst_hbm.at[pl.ds(base, CHUNK)], dstv[b],
                                isem[b]),
      )

    def idx_start(c, b):
      for d_ in idx_descs(c, b):
        d_.start()

    def idx_wait(c, b):
      for d_ in idx_descs(c, b):
        d_.wait()

    def start_gather(b):
      pltpu.make_async_copy(feats_hbm.at[srcv[b]], rows[b], gsem[b]).start()

    def wait_gather(b):
      pltpu.make_async_copy(feats_hbm.at[srcv[b]], rows[b], gsem[b]).wait()

    def scatter(b):
      pltpu.sync_copy(rows[b], acc.at[dstv[b]], add=True)

    def deg_update(b):
      if with_deg:
        for j in range(CHUNK // LANES):
          idx = dstv[b][pl.ds(j * LANES, LANES)]
          plsc.addupdate_scatter(degloc, [idx], ones16)

    idx_start(0, 0)
    idx_start(1, 1)
    idx_wait(0, 0)
    start_gather(0)

    @pl.loop(0, (nch - 1) // 2)
    def _(i):
      c = 2 * i
      idx_wait(c + 1, 1)
      wait_gather(0)
      start_gather(1)
      scatter(0)
      deg_update(0)
      idx_start(c + 2, 0)

      idx_wait(c + 2, 0)
      wait_gather(1)
      start_gather(0)
      scatter(1)
      deg_update(1)

      @pl.when(c + 3 < nch)
      def _():
        idx_start(c + 3, 1)

    wait_gather(0)
    scatter(0)
    deg_update(0)

    plsc.subcore_barrier()

    pltpu.sync_copy(acc.at[pl.ds(row0, nt)], out_hbm.at[cid, pl.ds(row0, nt)])

    @pl.when(tid == NS - 1)
    def _():
      pltpu.sync_copy(acc.at[pl.ds(NS * nt, rem)],
                      out_hbm.at[cid, pl.ds(NS * nt, rem)])

    if with_deg:
      pltpu.sync_copy(degloc.at[pl.ds(0, n)], deg_hbm.at[wid, 0])

  cp = pltpu.CompilerParams()
  if "needs_layout_passes" in pltpu.CompilerParams.__dataclass_fields__:
    cp = dataclasses.replace(cp, needs_layout_passes=False)
  if d % 128 != 0 or jnp.dtype(dtype) != jnp.dtype(jnp.float32):
    cp = dataclasses.replace(cp, use_tc_tiling_on_sc=False)
  return pl.kernel(body, out_type=out_type, mesh=mesh, scratch_types=scratch,
                   compiler_params=cp)


def _tc_layer1(agg_part, deg_part, w1, b1, w2):
  n = agg_part.shape[1]

  def body(aggp_ref, degp_ref, w1_ref, b1_ref, w2_ref, h2_ref, deg_ref):
    dp = degp_ref[...]
    deg = jnp.maximum(jnp.sum(dp, axis=(0, 1)), 1.0)
    deg_col = deg[:, None]
    deg_ref[...] = deg_col
    agg = (aggp_ref[0].astype(jnp.float32)
           + aggp_ref[1].astype(jnp.float32))
    h = agg / deg_col
    h = jnp.dot(h, w1_ref[...], preferred_element_type=jnp.float32)
    h = jnp.maximum(h + b1_ref[...], 0.0)
    h2 = jnp.dot(h, w2_ref[...], preferred_element_type=jnp.float32)
    h2_ref[...] = h2.astype(h2_ref.dtype)

  return pl.pallas_call(
      body,
      out_shape=[
          jax.ShapeDtypeStruct((n, w2.shape[1]), MSG_DTYPE),
          jax.ShapeDtypeStruct((n, 1), jnp.float32),
      ],
  )(agg_part, deg_part, w1, b1, w2)


def _tc_layer2(agg_part, deg_col, b2):
  n = agg_part.shape[1]

  def body(aggp_ref, deg_ref, b2_ref, out_ref):
    agg = aggp_ref[0].astype(jnp.float32) + aggp_ref[1].astype(jnp.float32)
    out_ref[...] = agg / deg_ref[...] + b2_ref[...]

  return pl.pallas_call(
      body,
      out_shape=jax.ShapeDtypeStruct((n, agg_part.shape[2]), jnp.float32),
  )(agg_part, deg_col, b2)


@jax.jit
def kernel(x, edge_index, W1, b1, W2, b2):
  n, d_in = x.shape
  e = edge_index.shape[1]
  ew = e // NW
  ew_pad = -(-ew // CHUNK) * CHUNK
  pad = ew_pad - ew

  def pad_edges(arr, fill_row):
    arr = arr.astype(jnp.int32).reshape(NW, ew)
    filler = jnp.broadcast_to(fill_row[None, :], (NW, pad))
    return jnp.concatenate([arr, filler], axis=1).reshape(-1)

  idx = jnp.arange(pad, dtype=jnp.int32)
  src = pad_edges(edge_index[0], idx % 64)
  dst = pad_edges(edge_index[1], n + idx % PAD_ROWS)
  e_pad = NW * ew_pad

  x_msg = x.astype(MSG_DTYPE)
  agg1_part, deg_part = _make_sc_aggregate(
      n, e_pad, d_in, True, MSG_DTYPE)(x_msg, src, dst)
  h2, deg_col = _tc_layer1(agg1_part, deg_part, W1, b1.reshape(1, -1), W2)
  (agg2_part,) = _make_sc_aggregate(
      n, e_pad, h2.shape[1], False, MSG_DTYPE)(h2, src, dst)
  return _tc_layer2(agg2_part, deg_col, b2.reshape(1, -1))


# speedup vs baseline: 2.0095x; 1.1946x over previous
import dataclasses

import jax
import jax.numpy as jnp
from jax import lax
from jax.experimental import pallas as pl
from jax.experimental.pallas import tpu as pltpu
from jax.experimental.pallas import tpu_sc as plsc

NC = 2
NS = 16
NW = NC * NS
LANES = 16
CHUNK = 128
PAD_ROWS = 16
MSG_DTYPE = jnp.bfloat16


def _make_sc_aggregate(n, e_pad, d, with_deg, dtype=jnp.float32):
  ew = e_pad // NW
  nch = ew // CHUNK
  assert ew % CHUNK == 0
  assert nch % 8 == 0
  nt = (n // NS) // 8 * 8
  rem = n - NS * nt
  mesh = plsc.VectorSubcoreMesh(core_axis_name="c", subcore_axis_name="s")

  out_type = [jax.ShapeDtypeStruct((NC, n, d), dtype)]
  if with_deg:
    out_type.append(jax.ShapeDtypeStruct((NW, 1, n), jnp.float32))

  R = 4
  M = 8
  scratch = (
      [pltpu.VMEM((CHUNK,), jnp.int32)] * M
      + [pltpu.VMEM((CHUNK,), jnp.int32)] * M
      + [pltpu.VMEM((CHUNK, d), dtype)] * R
      + [pltpu.VMEM_SHARED((n + PAD_ROWS, d), dtype)]
      + [pltpu.SemaphoreType.DMA] * R
      + [pltpu.SemaphoreType.DMA] * R
      + [pltpu.SemaphoreType.DMA] * M
  )
  if with_deg:
    scratch.append(
        pltpu.VMEM((n + PAD_ROWS,), jnp.float32))

  def body(feats_hbm, src_hbm, dst_hbm, *refs):
    refs = list(refs)
    out_hbm = refs.pop(0)
    deg_hbm = refs.pop(0) if with_deg else None
    degloc = refs.pop(-1) if with_deg else None
    srcv = [refs.pop(0) for _ in range(M)]
    dstv = [refs.pop(0) for _ in range(M)]
    rows = [refs.pop(0) for _ in range(R)]
    acc = refs.pop(0)
    gsem = [refs.pop(0) for _ in range(R)]
    ssem = [refs.pop(0) for _ in range(R)]
    isem = [refs.pop(0) for _ in range(M)]
    assert not refs
    rows0 = rows[0]

    cid = lax.axis_index("c")
    tid = lax.axis_index("s")
    wid = cid * NS + tid

    zero16 = jnp.zeros((LANES,), jnp.float32)
    vl = LANES * (4 // jnp.dtype(dtype).itemsize)
    zmsg = jnp.zeros((vl,), dtype)

    @pl.loop(0, CHUNK)
    def _(r):
      for k in range(d // vl):
        rows0[r, pl.ds(k * vl, vl)] = zmsg

    row0 = tid * nt
    for j in range(nt // CHUNK):
      pltpu.sync_copy(rows0, acc.at[pl.ds(row0 + j * CHUNK, CHUNK)])
    tail = nt % CHUNK
    if tail:
      pltpu.sync_copy(rows0.at[pl.ds(0, tail)],
                      acc.at[pl.ds(row0 + nt - tail, tail)])

    @pl.when(tid == NS - 1)
    def _():
      pltpu.sync_copy(rows0.at[pl.ds(0, rem)], acc.at[pl.ds(NS * nt, rem)])

    if with_deg:
      @pl.loop(0, n // LANES)
      def _(i):
        degloc[pl.ds(i * LANES, LANES)] = zero16

    plsc.subcore_barrier()

    ones16 = jnp.full((LANES,), 1.0, jnp.float32)
    base_w = wid * ew

    def idx_descs(c, m):
      base = base_w + c * CHUNK
      return (
          pltpu.make_async_copy(src_hbm.at[pl.ds(base, CHUNK)], srcv[m],
                                isem[m]),
          pltpu.make_async_copy(dst_hbm.at[pl.ds(base, CHUNK)], dstv[m],
                                isem[m]),
      )

    def idx_start(c, m):
      for d_ in idx_descs(c, m):
        d_.start()

    def idx_wait(c, m):
      for d_ in idx_descs(c, m):
        d_.wait()

    def gather_descr(m, b):
      return pltpu.make_async_copy(feats_hbm.at[srcv[m]], rows[b], gsem[b])

    def scatter_descr(m, b):
      return pltpu.make_async_copy(rows[b], acc.at[dstv[m]], ssem[b])

    def deg_update(m):
      if with_deg:
        for j in range(CHUNK // LANES):
          idx = dstv[m][pl.ds(j * LANES, LANES)]
          plsc.addupdate_scatter(degloc, [idx], ones16)

    for j in range(min(6, nch)):
      idx_start(j, j % M)
    idx_wait(0, 0)
    idx_wait(1, 1)
    gather_descr(0, 0).start()
    gather_descr(1, 1).start()

    def step(c, j):
      b = j % R
      m = j % M

      @pl.when(c + 2 < nch)
      def _():
        idx_wait(c + 2, (m + 2) % M)

      gather_descr(m, b).wait()
      scatter_descr(m, b).start(add=True)
      deg_update(m)

      @pl.when(c >= 2)
      def _():
        scatter_descr((m + 2) % M, (b + 2) % R).wait()

      @pl.when(c + 2 < nch)
      def _():
        gather_descr((m + 2) % M, (b + 2) % R).start()

      @pl.when(c + 6 < nch)
      def _():
        idx_start(c + 6, (m + 6) % M)

    assert nch % M == 0
    @pl.loop(0, nch // M)
    def _(i):
      for j in range(M):
        step(M * i + j, j)

    scatter_descr((nch - 2) % M, (nch - 2) % R).wait()
    scatter_descr((nch - 1) % M, (nch - 1) % R).wait()

    plsc.subcore_barrier()

    pltpu.sync_copy(acc.at[pl.ds(row0, nt)], out_hbm.at[cid, pl.ds(row0, nt)])

    @pl.when(tid == NS - 1)
    def _():
      pltpu.sync_copy(acc.at[pl.ds(NS * nt, rem)],
                      out_hbm.at[cid, pl.ds(NS * nt, rem)])

    if with_deg:
      pltpu.sync_copy(degloc.at[pl.ds(0, n)], deg_hbm.at[wid, 0])

  cp = pltpu.CompilerParams()
  if "needs_layout_passes" in pltpu.CompilerParams.__dataclass_fields__:
    cp = dataclasses.replace(cp, needs_layout_passes=False)
  if d % 128 != 0 or jnp.dtype(dtype) != jnp.dtype(jnp.float32):
    cp = dataclasses.replace(cp, use_tc_tiling_on_sc=False)
  return pl.kernel(body, out_type=out_type, mesh=mesh, scratch_types=scratch,
                   compiler_params=cp)


def _tc_layer1(agg_part, deg_part, w1, b1, w2):
  n = agg_part.shape[1]

  def body(aggp_ref, degp_ref, w1_ref, b1_ref, w2_ref, h2_ref, deg_ref):
    dp = degp_ref[...]
    deg = jnp.maximum(jnp.sum(dp, axis=(0, 1)), 1.0)
    deg_col = deg[:, None]
    deg_ref[...] = deg_col
    agg = (aggp_ref[0].astype(jnp.float32)
           + aggp_ref[1].astype(jnp.float32))
    h = agg / deg_col
    h = jnp.dot(h, w1_ref[...], preferred_element_type=jnp.float32)
    h = jnp.maximum(h + b1_ref[...], 0.0)
    h2 = jnp.dot(h, w2_ref[...], preferred_element_type=jnp.float32)
    h2_ref[...] = h2.astype(h2_ref.dtype)

  return pl.pallas_call(
      body,
      out_shape=[
          jax.ShapeDtypeStruct((n, w2.shape[1]), MSG_DTYPE),
          jax.ShapeDtypeStruct((n, 1), jnp.float32),
      ],
  )(agg_part, deg_part, w1, b1, w2)


def _tc_layer2(agg_part, deg_col, b2):
  n = agg_part.shape[1]

  def body(aggp_ref, deg_ref, b2_ref, out_ref):
    agg = aggp_ref[0].astype(jnp.float32) + aggp_ref[1].astype(jnp.float32)
    out_ref[...] = agg / deg_ref[...] + b2_ref[...]

  return pl.pallas_call(
      body,
      out_shape=jax.ShapeDtypeStruct((n, agg_part.shape[2]), jnp.float32),
  )(agg_part, deg_col, b2)


@jax.jit
def kernel(x, edge_index, W1, b1, W2, b2):
  n, d_in = x.shape
  e = edge_index.shape[1]
  ew = e // NW
  ew_pad = -(-ew // (8 * CHUNK)) * (8 * CHUNK)
  pad = ew_pad - ew

  def pad_edges(arr, fill_row):
    arr = arr.astype(jnp.int32).reshape(NW, ew)
    filler = jnp.broadcast_to(fill_row[None, :], (NW, pad))
    return jnp.concatenate([arr, filler], axis=1).reshape(-1)

  idx = jnp.arange(pad, dtype=jnp.int32)
  src = pad_edges(edge_index[0], idx % 64)
  dst = pad_edges(edge_index[1], n + idx % PAD_ROWS)
  e_pad = NW * ew_pad

  x_msg = x.astype(MSG_DTYPE)
  agg1_part, deg_part = _make_sc_aggregate(
      n, e_pad, d_in, True, MSG_DTYPE)(x_msg, src, dst)
  h2, deg_col = _tc_layer1(agg1_part, deg_part, W1, b1.reshape(1, -1), W2)
  (agg2_part,) = _make_sc_aggregate(
      n, e_pad, h2.shape[1], False, MSG_DTYPE)(h2, src, dst)
  return _tc_layer2(agg2_part, deg_col, b2.reshape(1, -1))
